# baseline (speedup 1.0000x reference)
import jax
import jax.numpy as jnp
from jax.experimental import pallas as pl
from jax.experimental.pallas import tpu as pltpu


def _transpose_body(x_ref, o_ref):
    o_ref[...] = x_ref[...].T


def kernel(input):
    b, e, h, w = input.shape
    hw = h * w
    x = input.reshape(b * e, hw)
    out = pl.pallas_call(
        _transpose_body,
        grid=(b,),
        in_specs=[pl.BlockSpec((e, hw), lambda i: (i, 0))],
        out_specs=pl.BlockSpec((hw, e), lambda i: (i, 0)),
        out_shape=jax.ShapeDtypeStruct((b * hw, e), x.dtype),
        compiler_params=pltpu.CompilerParams(
            dimension_semantics=("parallel",),
        ),
    )(x)
    length = jnp.full((b,), True, dtype=bool)
    return (out.reshape(b, hw, e), length)


# speedup vs baseline: 1.7458x; 1.7458x over previous
import jax
import jax.numpy as jnp
from jax.experimental import pallas as pl
from jax.experimental.pallas import tpu as pltpu


def _transpose_body(x_ref, o_ref):
    o_ref[0] = x_ref[0].T


def kernel(input):
    b, e, h, w = input.shape
    hw = h * w
    eb = 256
    x = input.reshape(b, e, hw)
    out = pl.pallas_call(
        _transpose_body,
        grid=(b, e // eb),
        in_specs=[pl.BlockSpec((1, eb, hw), lambda i, j: (i, j, 0))],
        out_specs=pl.BlockSpec((1, hw, eb), lambda i, j: (i, 0, j)),
        out_shape=jax.ShapeDtypeStruct((b, hw, e), x.dtype),
        compiler_params=pltpu.CompilerParams(
            dimension_semantics=("parallel", "parallel"),
        ),
    )(x)
    length = jnp.full((b,), True, dtype=bool)
    return (out, length)


# speedup vs baseline: 2.2751x; 1.3032x over previous
import jax
import jax.numpy as jnp
from jax.experimental import pallas as pl
from jax.experimental.pallas import tpu as pltpu


def _transpose_body(x_ref, o_ref):
    o_ref[0] = x_ref[0].T


def kernel(input):
    b, e, h, w = input.shape
    hw = h * w
    x = input.reshape(b, e, hw)
    out = pl.pallas_call(
        _transpose_body,
        grid=(b,),
        in_specs=[pl.BlockSpec((1, e, hw), lambda i: (i, 0, 0))],
        out_specs=pl.BlockSpec((1, hw, e), lambda i: (i, 0, 0)),
        out_shape=jax.ShapeDtypeStruct((b, hw, e), x.dtype),
        compiler_params=pltpu.CompilerParams(
            dimension_semantics=("parallel",),
        ),
    )(x)
    length = jnp.full((b,), True, dtype=bool)
    return (out, length)


# speedup vs baseline: 2.5226x; 1.1088x over previous
import jax
import jax.numpy as jnp
from jax.experimental import pallas as pl
from jax.experimental.pallas import tpu as pltpu

_NBUF = 4


def _pipeline_body(x_hbm, o_hbm, in_buf, out_buf, in_sem, out_sem):
    b = x_hbm.shape[0]

    def in_copy(i, slot):
        return pltpu.make_async_copy(x_hbm.at[i], in_buf.at[slot], in_sem.at[slot])

    def out_copy(i, slot):
        return pltpu.make_async_copy(out_buf.at[slot], o_hbm.at[i], out_sem.at[slot])

    for s in range(_NBUF):
        in_copy(s, s).start()
    for i in range(b):
        slot = i % _NBUF
        in_copy(i, slot).wait()
        if i >= _NBUF:
            out_copy(i - _NBUF, slot).wait()
        out_buf[slot] = in_buf[slot].T
        out_copy(i, slot).start()
        nxt = i + _NBUF
        if nxt < b:
            in_copy(nxt, slot).start()
    for i in range(b - _NBUF, b):
        out_copy(i, i % _NBUF).wait()


def kernel(input):
    b, e, h, w = input.shape
    hw = h * w
    x = input.reshape(b, e, hw)
    out = pl.pallas_call(
        _pipeline_body,
        in_specs=[pl.BlockSpec(memory_space=pltpu.MemorySpace.HBM)],
        out_specs=pl.BlockSpec(memory_space=pltpu.MemorySpace.HBM),
        out_shape=jax.ShapeDtypeStruct((b, hw, e), x.dtype),
        scratch_shapes=[
            pltpu.VMEM((_NBUF, e, hw), x.dtype),
            pltpu.VMEM((_NBUF, hw, e), x.dtype),
            pltpu.SemaphoreType.DMA((_NBUF,)),
            pltpu.SemaphoreType.DMA((_NBUF,)),
        ],
    )(x)
    length = jnp.full((b,), True, dtype=bool)
    return (out, length)
